# Initial kernel scaffold; baseline (speedup 1.0000x reference)
import jax
import jax.numpy as jnp
from jax.experimental import pallas as pl


def kernel(last_hidden_state, labeled_input_ids, W_w, W_b, V_w, V_b):
    raise NotImplementedError("write your pallas kernel here")


# speedup vs baseline: 5.9063x; 5.9063x over previous
import functools

import jax
import jax.numpy as jnp
from jax.experimental import pallas as pl
from jax.experimental.pallas import tpu as pltpu

NSEG = 128
S_BLK = 512
H_BLK = 512


def _seg_mean_kernel(labels_ref, x_ref, feat_ref, sum_scr, cnt_scr):
    s = pl.program_id(1)
    ns = pl.num_programs(1)

    @pl.when(s == 0)
    def _init():
        sum_scr[...] = jnp.zeros_like(sum_scr)
        cnt_scr[...] = jnp.zeros_like(cnt_scr)

    chunk = labels_ref[0, 0, pl.ds(s * S_BLK, S_BLK)]
    ids = chunk.reshape(1, S_BLK)
    seg = jax.lax.broadcasted_iota(jnp.int32, (NSEG, S_BLK), 0)
    p = (seg == ids).astype(jnp.float32)
    x = x_ref[0]
    sum_scr[...] += jnp.dot(p, x, preferred_element_type=jnp.float32)
    cnt_scr[...] += jnp.sum(p, axis=1, keepdims=True)

    @pl.when(s == ns - 1)
    def _fin():
        feat_ref[0] = sum_scr[...] / jnp.maximum(cnt_scr[...], 1e-12)


def _attend_kernel(feat_ref, labels_ref, ww_ref, wb_ref, vw_ref, out_ref,
                   score_scr):
    h = pl.program_id(1)
    nh = pl.num_programs(1)

    @pl.when(h == 0)
    def _init():
        score_scr[...] = jnp.zeros_like(score_scr)

    f = feat_ref[0]
    att = jax.lax.dot_general(
        f, ww_ref[...], (((1,), (1,)), ((), ())),
        preferred_element_type=jnp.float32)
    att = jnp.tanh(att + wb_ref[...])
    score_scr[...] += jax.lax.dot_general(
        att, vw_ref[...], (((1,), (1,)), ((), ())),
        preferred_element_type=jnp.float32)

    @pl.when(h == nh - 1)
    def _fin():
        s_len = labels_ref.shape[-1]
        n = labels_ref[0, 0, s_len - 1] + 1
        seg = jax.lax.broadcasted_iota(jnp.int32, (NSEG, 1), 0)
        valid = jnp.logical_and(seg >= 1, seg < n)
        score = jnp.where(valid, score_scr[...], jnp.float32(-10000.0))
        m = jnp.max(score, axis=0, keepdims=True)
        e = jnp.exp(score - m)
        w = e / jnp.sum(e, axis=0, keepdims=True)
        w = w * valid.astype(jnp.float32)
        out_ref[0] = jax.lax.dot_general(
            w, f, (((0,), (0,)), ((), ())),
            preferred_element_type=jnp.float32)


@functools.partial(jax.jit, static_argnames=("interpret",))
def _run(last_hidden_state, labels3, W_w, W_b2, V_w, interpret=False):
    B, S, D = last_hidden_state.shape
    H = W_w.shape[0]

    feat = pl.pallas_call(
        _seg_mean_kernel,
        grid=(B, S // S_BLK),
        in_specs=[
            pl.BlockSpec((1, 1, S), lambda i, s: (i, 0, 0)),
            pl.BlockSpec((1, S_BLK, D), lambda i, s: (i, s, 0)),
        ],
        out_specs=pl.BlockSpec((1, NSEG, D), lambda i, s: (i, 0, 0)),
        out_shape=jax.ShapeDtypeStruct((B, NSEG, D), jnp.float32),
        scratch_shapes=[
            pltpu.VMEM((NSEG, D), jnp.float32),
            pltpu.VMEM((NSEG, 1), jnp.float32),
        ],
        interpret=interpret,
    )(labels3, last_hidden_state)

    ctx = pl.pallas_call(
        _attend_kernel,
        grid=(B, H // H_BLK),
        in_specs=[
            pl.BlockSpec((1, NSEG, D), lambda i, h: (i, 0, 0)),
            pl.BlockSpec((1, 1, S), lambda i, h: (i, 0, 0)),
            pl.BlockSpec((H_BLK, D), lambda i, h: (h, 0)),
            pl.BlockSpec((1, H_BLK), lambda i, h: (0, h)),
            pl.BlockSpec((1, H_BLK), lambda i, h: (0, h)),
        ],
        out_specs=pl.BlockSpec((1, 1, D), lambda i, h: (i, 0, 0)),
        out_shape=jax.ShapeDtypeStruct((B, 1, D), jnp.float32),
        scratch_shapes=[pltpu.VMEM((NSEG, 1), jnp.float32)],
        interpret=interpret,
    )(feat, labels3, W_w, W_b2, V_w)
    return ctx.reshape(B, D)


def kernel(last_hidden_state, labeled_input_ids, W_w, W_b, V_w, V_b):
    B, S, D = last_hidden_state.shape
    H = W_w.shape[0]
    labels3 = labeled_input_ids.astype(jnp.int32).reshape(B, 1, S)
    return _run(last_hidden_state, labels3, W_w, W_b.reshape(1, H), V_w)


# speedup vs baseline: 7.9065x; 1.3387x over previous
import functools

import jax
import jax.numpy as jnp
from jax.experimental import pallas as pl
from jax.experimental.pallas import tpu as pltpu

NSEG = 128
S_BLK = 512


def _fused_kernel(labels_ref, x_ref, ww_ref, wb_ref, vw_ref, out_ref,
                  sum_scr, cnt_scr):
    s = pl.program_id(1)
    ns = pl.num_programs(1) - 1

    @pl.when(s == 0)
    def _init():
        sum_scr[...] = jnp.zeros_like(sum_scr)
        cnt_scr[...] = jnp.zeros_like(cnt_scr)

    @pl.when(s < ns)
    def _accum():
        chunk = labels_ref[0, 0, pl.ds(s * S_BLK, S_BLK)]
        ids = chunk.reshape(1, S_BLK)
        seg = jax.lax.broadcasted_iota(jnp.int32, (NSEG, S_BLK), 0)
        p = (seg == ids).astype(jnp.float32)
        sum_scr[...] += jnp.dot(p, x_ref[0],
                                preferred_element_type=jnp.float32)
        cnt_scr[...] += jnp.sum(p, axis=1, keepdims=True)

    @pl.when(s == ns)
    def _attend():
        f = sum_scr[...] / jnp.maximum(cnt_scr[...], 1e-12)
        att = jax.lax.dot_general(
            f, ww_ref[...], (((1,), (1,)), ((), ())),
            preferred_element_type=jnp.float32)
        att = jnp.tanh(att + wb_ref[...])
        score = jax.lax.dot_general(
            att, vw_ref[...], (((1,), (1,)), ((), ())),
            preferred_element_type=jnp.float32)
        s_len = labels_ref.shape[-1]
        n = labels_ref[0, 0, s_len - 1] + 1
        segc = jax.lax.broadcasted_iota(jnp.int32, (NSEG, 1), 0)
        valid = jnp.logical_and(segc >= 1, segc < n)
        score = jnp.where(valid, score, jnp.float32(-10000.0))
        m = jnp.max(score, axis=0, keepdims=True)
        e = jnp.exp(score - m)
        w = e / jnp.sum(e, axis=0, keepdims=True)
        w = w * valid.astype(jnp.float32)
        out_ref[0] = jax.lax.dot_general(
            w, f, (((0,), (0,)), ((), ())),
            preferred_element_type=jnp.float32)


@functools.partial(jax.jit, static_argnames=("interpret",))
def _run(last_hidden_state, labels3, W_w, W_b2, V_w, interpret=False):
    B, S, D = last_hidden_state.shape
    H = W_w.shape[0]
    ns = S // S_BLK

    ctx = pl.pallas_call(
        _fused_kernel,
        grid=(B, ns + 1),
        in_specs=[
            pl.BlockSpec((1, 1, S), lambda i, s: (i, 0, 0)),
            pl.BlockSpec((1, S_BLK, D),
                         lambda i, s: (i, jnp.minimum(s, ns - 1), 0)),
            pl.BlockSpec((H, D), lambda i, s: (0, 0)),
            pl.BlockSpec((1, H), lambda i, s: (0, 0)),
            pl.BlockSpec((1, H), lambda i, s: (0, 0)),
        ],
        out_specs=pl.BlockSpec((1, 1, D), lambda i, s: (i, 0, 0)),
        out_shape=jax.ShapeDtypeStruct((B, 1, D), jnp.float32),
        scratch_shapes=[
            pltpu.VMEM((NSEG, D), jnp.float32),
            pltpu.VMEM((NSEG, 1), jnp.float32),
        ],
        compiler_params=pltpu.CompilerParams(
            dimension_semantics=("parallel", "arbitrary")),
        interpret=interpret,
    )(labels3, last_hidden_state, W_w, W_b2, V_w)
    return ctx.reshape(B, D)


def kernel(last_hidden_state, labeled_input_ids, W_w, W_b, V_w, V_b):
    B, S, D = last_hidden_state.shape
    H = W_w.shape[0]
    labels3 = labeled_input_ids.astype(jnp.int32).reshape(B, 1, S)
    return _run(last_hidden_state, labels3, W_w, W_b.reshape(1, H), V_w)


# speedup vs baseline: 8.7597x; 1.1079x over previous
import functools

import jax
import jax.numpy as jnp
from jax.experimental import pallas as pl
from jax.experimental.pallas import tpu as pltpu

NSEG = 128
S_BLK = 1024


def _fused_kernel(labels_ref, x_ref, ww_ref, wb_ref, vw_ref, out_ref,
                  sum_scr, cnt_scr):
    s = pl.program_id(1)
    ns = pl.num_programs(1) - 1

    @pl.when(s == 0)
    def _init():
        sum_scr[...] = jnp.zeros_like(sum_scr)
        cnt_scr[...] = jnp.zeros_like(cnt_scr)

    @pl.when(s < ns)
    def _accum():
        chunk = labels_ref[0, 0, pl.ds(s * S_BLK, S_BLK)]
        ids = chunk.reshape(1, S_BLK)
        seg = jax.lax.broadcasted_iota(jnp.int32, (NSEG, S_BLK), 0)
        p = (seg == ids).astype(jnp.bfloat16)
        sum_scr[...] += jnp.dot(p, x_ref[0].astype(jnp.bfloat16),
                                preferred_element_type=jnp.float32)
        cnt_scr[...] += jnp.sum(p.astype(jnp.float32), axis=1, keepdims=True)

    @pl.when(s == ns)
    def _attend():
        rec = 1.0 / jnp.maximum(cnt_scr[...], 1e-12)
        f = sum_scr[...] * rec
        att = jax.lax.dot_general(
            f.astype(jnp.bfloat16), ww_ref[...].astype(jnp.bfloat16),
            (((1,), (1,)), ((), ())),
            preferred_element_type=jnp.float32)
        att = jnp.tanh(att + wb_ref[...])
        score = jax.lax.dot_general(
            att, vw_ref[...], (((1,), (1,)), ((), ())),
            preferred_element_type=jnp.float32)
        s_len = labels_ref.shape[-1]
        n = labels_ref[0, 0, s_len - 1] + 1
        segc = jax.lax.broadcasted_iota(jnp.int32, (NSEG, 1), 0)
        valid = jnp.logical_and(segc >= 1, segc < n)
        score = jnp.where(valid, score, jnp.float32(-10000.0))
        m = jnp.max(score, axis=0, keepdims=True)
        e = jnp.exp(score - m)
        w = e / jnp.sum(e, axis=0, keepdims=True)
        w = w * valid.astype(jnp.float32)
        out_ref[0] = jax.lax.dot_general(
            w, f, (((0,), (0,)), ((), ())),
            preferred_element_type=jnp.float32)


@functools.partial(jax.jit, static_argnames=("interpret",))
def _run(last_hidden_state, labels3, W_w, W_b2, V_w, interpret=False):
    B, S, D = last_hidden_state.shape
    H = W_w.shape[0]
    ns = S // S_BLK

    ctx = pl.pallas_call(
        _fused_kernel,
        grid=(B, ns + 1),
        in_specs=[
            pl.BlockSpec((1, 1, S), lambda i, s: (i, 0, 0)),
            pl.BlockSpec((1, S_BLK, D),
                         lambda i, s: (i, jnp.minimum(s, ns - 1), 0)),
            pl.BlockSpec((H, D), lambda i, s: (0, 0)),
            pl.BlockSpec((1, H), lambda i, s: (0, 0)),
            pl.BlockSpec((1, H), lambda i, s: (0, 0)),
        ],
        out_specs=pl.BlockSpec((1, 1, D), lambda i, s: (i, 0, 0)),
        out_shape=jax.ShapeDtypeStruct((B, 1, D), jnp.float32),
        scratch_shapes=[
            pltpu.VMEM((NSEG, D), jnp.float32),
            pltpu.VMEM((NSEG, 1), jnp.float32),
        ],
        compiler_params=pltpu.CompilerParams(
            dimension_semantics=("parallel", "arbitrary")),
        interpret=interpret,
    )(labels3, last_hidden_state, W_w, W_b2, V_w)
    return ctx.reshape(B, D)


def kernel(last_hidden_state, labeled_input_ids, W_w, W_b, V_w, V_b):
    B, S, D = last_hidden_state.shape
    H = W_w.shape[0]
    labels3 = labeled_input_ids.astype(jnp.int32).reshape(B, 1, S)
    return _run(last_hidden_state, labels3, W_w, W_b.reshape(1, H), V_w)


# speedup vs baseline: 9.3974x; 1.0728x over previous
import functools

import jax
import jax.numpy as jnp
from jax.experimental import pallas as pl
from jax.experimental.pallas import tpu as pltpu

NSEG = 128
S_BLK = 2048


def _fused_kernel(labels_ref, x_ref, ww_ref, wb_ref, vw_ref, out_ref,
                  sum_scr, cnt_scr):
    s = pl.program_id(1)
    ns = pl.num_programs(1) - 1

    @pl.when(s == 0)
    def _init():
        sum_scr[...] = jnp.zeros_like(sum_scr)
        cnt_scr[...] = jnp.zeros_like(cnt_scr)

    @pl.when(s < ns)
    def _accum():
        chunk = labels_ref[0, 0, pl.ds(s * S_BLK, S_BLK)]
        ids = chunk.reshape(1, S_BLK)
        seg = jax.lax.broadcasted_iota(jnp.int32, (NSEG, S_BLK), 0)
        p = (seg == ids).astype(jnp.bfloat16)
        sum_scr[...] += jnp.dot(p, x_ref[0].astype(jnp.bfloat16),
                                preferred_element_type=jnp.float32)
        cnt_scr[...] += jnp.sum(p.astype(jnp.float32), axis=1, keepdims=True)

    @pl.when(s == ns)
    def _attend():
        rec = 1.0 / jnp.maximum(cnt_scr[...], 1e-12)
        f = sum_scr[...] * rec
        att = jax.lax.dot_general(
            f.astype(jnp.bfloat16), ww_ref[...].astype(jnp.bfloat16),
            (((1,), (1,)), ((), ())),
            preferred_element_type=jnp.float32)
        att = jnp.tanh(att + wb_ref[...])
        score = jax.lax.dot_general(
            att, vw_ref[...], (((1,), (1,)), ((), ())),
            preferred_element_type=jnp.float32)
        s_len = labels_ref.shape[-1]
        n = labels_ref[0, 0, s_len - 1] + 1
        segc = jax.lax.broadcasted_iota(jnp.int32, (NSEG, 1), 0)
        valid = jnp.logical_and(segc >= 1, segc < n)
        score = jnp.where(valid, score, jnp.float32(-10000.0))
        m = jnp.max(score, axis=0, keepdims=True)
        e = jnp.exp(score - m)
        w = e / jnp.sum(e, axis=0, keepdims=True)
        w = w * valid.astype(jnp.float32)
        out_ref[0] = jax.lax.dot_general(
            w, f, (((0,), (0,)), ((), ())),
            preferred_element_type=jnp.float32)


@functools.partial(jax.jit, static_argnames=("interpret",))
def _run(last_hidden_state, labels3, W_w, W_b2, V_w, interpret=False):
    B, S, D = last_hidden_state.shape
    H = W_w.shape[0]
    ns = S // S_BLK

    ctx = pl.pallas_call(
        _fused_kernel,
        grid=(B, ns + 1),
        in_specs=[
            pl.BlockSpec((1, 1, S), lambda i, s: (i, 0, 0)),
            pl.BlockSpec((1, S_BLK, D),
                         lambda i, s: (i, jnp.minimum(s, ns - 1), 0)),
            pl.BlockSpec((H, D), lambda i, s: (0, 0)),
            pl.BlockSpec((1, H), lambda i, s: (0, 0)),
            pl.BlockSpec((1, H), lambda i, s: (0, 0)),
        ],
        out_specs=pl.BlockSpec((1, 1, D), lambda i, s: (i, 0, 0)),
        out_shape=jax.ShapeDtypeStruct((B, 1, D), jnp.float32),
        scratch_shapes=[
            pltpu.VMEM((NSEG, D), jnp.float32),
            pltpu.VMEM((NSEG, 1), jnp.float32),
        ],
        compiler_params=pltpu.CompilerParams(
            dimension_semantics=("parallel", "arbitrary")),
        interpret=interpret,
    )(labels3, last_hidden_state, W_w, W_b2, V_w)
    return ctx.reshape(B, D)


def kernel(last_hidden_state, labeled_input_ids, W_w, W_b, V_w, V_b):
    B, S, D = last_hidden_state.shape
    H = W_w.shape[0]
    labels3 = labeled_input_ids.astype(jnp.int32).reshape(B, 1, S)
    return _run(last_hidden_state, labels3, W_w, W_b.reshape(1, H), V_w)


# speedup vs baseline: 10.3498x; 1.1013x over previous
import functools

import jax
import jax.numpy as jnp
from jax.experimental import pallas as pl
from jax.experimental.pallas import tpu as pltpu

NSEG = 128


def _fused_kernel(labels_ref, x_ref, ww_ref, wb_ref, vw_ref, out_ref):
    s_len = labels_ref.shape[-1]
    ids = labels_ref[0, 0, :].reshape(1, s_len)
    seg = jax.lax.broadcasted_iota(jnp.int32, (NSEG, s_len), 0)
    onehot = seg == ids
    p = onehot.astype(jnp.bfloat16)
    sums = jnp.dot(p, x_ref[0].astype(jnp.bfloat16),
                   preferred_element_type=jnp.float32)
    cnt = jnp.sum(onehot.astype(jnp.float32), axis=1, keepdims=True)
    rec = 1.0 / jnp.maximum(cnt, 1e-12)
    f = sums * rec
    att = jax.lax.dot_general(
        f.astype(jnp.bfloat16), ww_ref[...],
        (((1,), (1,)), ((), ())),
        preferred_element_type=jnp.float32)
    att = jnp.tanh(att + wb_ref[...])
    score = jax.lax.dot_general(
        att, vw_ref[...], (((1,), (1,)), ((), ())),
        preferred_element_type=jnp.float32)
    n = labels_ref[0, 0, s_len - 1] + 1
    segc = jax.lax.broadcasted_iota(jnp.int32, (NSEG, 1), 0)
    valid = jnp.logical_and(segc >= 1, segc < n)
    score = jnp.where(valid, score, jnp.float32(-10000.0))
    m = jnp.max(score, axis=0, keepdims=True)
    e = jnp.exp(score - m)
    w = e / jnp.sum(e, axis=0, keepdims=True)
    w = w * valid.astype(jnp.float32)
    out_ref[0] = jax.lax.dot_general(
        w, f, (((0,), (0,)), ((), ())),
        preferred_element_type=jnp.float32)


@functools.partial(jax.jit, static_argnames=("interpret",))
def _run(last_hidden_state, labels3, W_w, W_b2, V_w, interpret=False):
    B, S, D = last_hidden_state.shape
    H = W_w.shape[0]

    ctx = pl.pallas_call(
        _fused_kernel,
        grid=(B,),
        in_specs=[
            pl.BlockSpec((1, 1, S), lambda i: (i, 0, 0)),
            pl.BlockSpec((1, S, D), lambda i: (i, 0, 0)),
            pl.BlockSpec((H, D), lambda i: (0, 0)),
            pl.BlockSpec((1, H), lambda i: (0, 0)),
            pl.BlockSpec((1, H), lambda i: (0, 0)),
        ],
        out_specs=pl.BlockSpec((1, 1, D), lambda i: (i, 0, 0)),
        out_shape=jax.ShapeDtypeStruct((B, 1, D), jnp.float32),
        compiler_params=pltpu.CompilerParams(
            dimension_semantics=("arbitrary",)),
        interpret=interpret,
    )(labels3, last_hidden_state, W_w.astype(jnp.bfloat16), W_b2, V_w)
    return ctx.reshape(B, D)


def kernel(last_hidden_state, labeled_input_ids, W_w, W_b, V_w, V_b):
    B, S, D = last_hidden_state.shape
    H = W_w.shape[0]
    labels3 = labeled_input_ids.astype(jnp.int32).reshape(B, 1, S)
    return _run(last_hidden_state, labels3, W_w, W_b.reshape(1, H), V_w)


# speedup vs baseline: 11.6709x; 1.1276x over previous
import functools

import jax
import jax.numpy as jnp
from jax.experimental import pallas as pl
from jax.experimental.pallas import tpu as pltpu

NSEG = 128


def _fused_kernel(labels_ref, x_ref, ww_ref, wb_ref, vw_ref, out_ref):
    s_len = labels_ref.shape[-1]
    ids = labels_ref[0, 0, :].reshape(1, s_len)
    seg = jax.lax.broadcasted_iota(jnp.int32, (NSEG, s_len), 0)
    onehot = seg == ids
    p = onehot.astype(jnp.bfloat16)
    sums = jnp.dot(p, x_ref[0].astype(jnp.bfloat16),
                   preferred_element_type=jnp.float32)
    cnt = jnp.sum(onehot.astype(jnp.float32), axis=1, keepdims=True)
    rec = 1.0 / jnp.maximum(cnt, 1e-12)
    f = sums * rec
    att = jax.lax.dot_general(
        f.astype(jnp.bfloat16), ww_ref[...].astype(jnp.bfloat16),
        (((1,), (1,)), ((), ())),
        preferred_element_type=jnp.float32)
    att = jnp.tanh(att + wb_ref[...])
    score = jax.lax.dot_general(
        att, vw_ref[...], (((1,), (1,)), ((), ())),
        preferred_element_type=jnp.float32)
    n = labels_ref[0, 0, s_len - 1] + 1
    segc = jax.lax.broadcasted_iota(jnp.int32, (NSEG, 1), 0)
    valid = jnp.logical_and(segc >= 1, segc < n)
    score = jnp.where(valid, score, jnp.float32(-10000.0))
    m = jnp.max(score, axis=0, keepdims=True)
    e = jnp.exp(score - m)
    w = e / jnp.sum(e, axis=0, keepdims=True)
    w = w * valid.astype(jnp.float32)
    out_ref[0] = jax.lax.dot_general(
        w, f, (((0,), (0,)), ((), ())),
        preferred_element_type=jnp.float32)


@functools.partial(jax.jit, static_argnames=("interpret",))
def _run(last_hidden_state, labels3, W_w, W_b2, V_w, interpret=False):
    B, S, D = last_hidden_state.shape
    H = W_w.shape[0]

    ctx = pl.pallas_call(
        _fused_kernel,
        grid=(B,),
        in_specs=[
            pl.BlockSpec((1, 1, S), lambda i: (i, 0, 0)),
            pl.BlockSpec((1, S, D), lambda i: (i, 0, 0)),
            pl.BlockSpec((H, D), lambda i: (0, 0)),
            pl.BlockSpec((1, H), lambda i: (0, 0)),
            pl.BlockSpec((1, H), lambda i: (0, 0)),
        ],
        out_specs=pl.BlockSpec((1, 1, D), lambda i: (i, 0, 0)),
        out_shape=jax.ShapeDtypeStruct((B, 1, D), jnp.float32),
        compiler_params=pltpu.CompilerParams(
            dimension_semantics=("arbitrary",)),
        interpret=interpret,
    )(labels3, last_hidden_state, W_w, W_b2, V_w)
    return ctx.reshape(B, D)


def kernel(last_hidden_state, labeled_input_ids, W_w, W_b, V_w, V_b):
    B, S, D = last_hidden_state.shape
    H = W_w.shape[0]
    labels3 = labeled_input_ids.astype(jnp.int32).reshape(B, 1, S)
    return _run(last_hidden_state, labels3, W_w, W_b.reshape(1, H), V_w)
